# baseline (speedup 1.0000x reference)
import functools

import jax
import jax.numpy as jnp
from jax import lax
from jax.experimental import pallas as pl
from jax.experimental.pallas import tpu as pltpu
from jax.experimental.pallas import tpu_sc as plsc

D = 64
SCALE = 8.0

_NC = 2
_NS = 16
_NW = _NC * _NS

_IDXW = 128
_CHUNK_ROWS = 8
_CHUNK = _CHUNK_ROWS * _IDXW


def _make_sc_kernel(B):
    n_idx_rows = B // _IDXW
    rows_per_w = n_idx_rows // _NW
    chunks_per_w = rows_per_w // _CHUNK_ROWS
    mesh = plsc.VectorSubcoreMesh(core_axis_name="c", subcore_axis_name="s")

    @functools.partial(
        pl.kernel,
        mesh=mesh,
        out_type=jax.ShapeDtypeStruct((B, D), jnp.float32),
        scratch_types=[
            pltpu.VMEM((_CHUNK_ROWS, _IDXW), jnp.int32),
            pltpu.VMEM((_CHUNK, D), jnp.float32),
            pltpu.SemaphoreType.DMA,
        ],
        compiler_params=pltpu.CompilerParams(use_tc_tiling_on_sc=False),
    )
    def k(idx_hbm, table_hbm, out_hbm, idx_v, rows_v, sem):
        wid = lax.axis_index("s") * _NC + lax.axis_index("c")
        row0 = wid * rows_per_w

        def chunk_body(g, carry):
            r = row0 + g * _CHUNK_ROWS
            pltpu.sync_copy(idx_hbm.at[pl.ds(r, _CHUNK_ROWS)], idx_v)
            copies = [
                pltpu.async_copy(
                    table_hbm.at[idx_v.at[j]],
                    rows_v.at[pl.ds(j * _IDXW, _IDXW)],
                    sem,
                )
                for j in range(_CHUNK_ROWS)
            ]
            for c in copies:
                c.wait()

            def scale_body(i, c2):
                for u in range(4):
                    rr = i * 4 + u
                    for j in range(D // 16):
                        sl = pl.ds(j * 16, 16)
                        rows_v[rr, sl] = rows_v[rr, sl] * SCALE
                return c2

            lax.fori_loop(0, _CHUNK // 4, scale_body, 0)
            pltpu.sync_copy(rows_v, out_hbm.at[pl.ds(r * _IDXW, _CHUNK)])
            return carry

        lax.fori_loop(0, chunks_per_w, chunk_body, 0)

    return k


def kernel(tokens, embedding):
    B = tokens.shape[0] * tokens.shape[1]
    idx = tokens.reshape(B // _IDXW, _IDXW).astype(jnp.int32)
    out = _make_sc_kernel(B)(idx, embedding)
    return out.reshape(*tokens.shape, D)


# speedup vs baseline: 1.0591x; 1.0591x over previous
import functools

import jax
import jax.numpy as jnp
from jax import lax
from jax.experimental import pallas as pl
from jax.experimental.pallas import tpu as pltpu
from jax.experimental.pallas import tpu_sc as plsc

D = 64
SCALE = 8.0

_NC = 2
_NS = 16
_NW = _NC * _NS

_IDXW = 128
_CROWS = 5
_CHUNK = _CROWS * _IDXW


def _make_sc_kernel(B):
    rows_per_w = B // _IDXW // _NW
    nchunks = rows_per_w // _CROWS
    mesh = plsc.VectorSubcoreMesh(core_axis_name="c", subcore_axis_name="s")

    @functools.partial(
        pl.kernel,
        mesh=mesh,
        out_type=jax.ShapeDtypeStruct((B, D), jnp.float32),
        scratch_types=[
            pltpu.VMEM((rows_per_w, _IDXW), jnp.int32),
            pltpu.VMEM((2, _CHUNK, D), jnp.float32),
            pltpu.SemaphoreType.DMA,
            pltpu.SemaphoreType.DMA,
            pltpu.SemaphoreType.DMA,
            pltpu.SemaphoreType.DMA,
        ],
        compiler_params=pltpu.CompilerParams(use_tc_tiling_on_sc=False),
    )
    def k(idx_hbm, table_hbm, out_hbm, idx_v, rows_v, g0, g1, w0, w1):
        wid = lax.axis_index("s") * _NC + lax.axis_index("c")
        row0 = wid * rows_per_w
        out0 = row0 * _IDXW
        gsem = (g0, g1)
        wsem = (w0, w1)

        def fire(g, b):
            for j in range(_CROWS):
                pltpu.async_copy(
                    table_hbm.at[idx_v.at[g * _CROWS + j]],
                    rows_v.at[b].at[pl.ds(j * _IDXW, _IDXW)],
                    gsem[b],
                )

        def wait_gather(b):
            pltpu.make_async_copy(
                out_hbm.at[pl.ds(0, _CHUNK)], rows_v.at[b], gsem[b]
            ).wait()

        def scale(b):
            def body(i, c):
                for u in range(8):
                    r = i * 8 + u
                    for j in range(D // 16):
                        sl = pl.ds(j * 16, 16)
                        rows_v[b, r, sl] = rows_v[b, r, sl] * SCALE
                return c

            lax.fori_loop(0, _CHUNK // 8, body, 0)

        def fire_wb(g, b):
            pltpu.async_copy(
                rows_v.at[b], out_hbm.at[pl.ds(out0 + g * _CHUNK, _CHUNK)],
                wsem[b],
            )

        def wait_wb(b):
            pltpu.make_async_copy(
                rows_v.at[b], out_hbm.at[pl.ds(0, _CHUNK)], wsem[b]
            ).wait()

        pltpu.sync_copy(idx_hbm.at[pl.ds(row0, rows_per_w)], idx_v)

        fire(0, 0)
        fire(1, 1)
        wait_gather(0)
        scale(0)
        fire_wb(0, 0)

        def outer(s, carry):
            for par in range(2):
                g = 1 + s * 2 + par
                b = (1 + par) % 2
                nb = 1 - b
                wait_wb(nb)
                fire(g + 1, nb)
                wait_gather(b)
                scale(b)
                fire_wb(g, b)
            return carry

        lax.fori_loop(0, (nchunks - 2) // 2, outer, 0)

        gl = nchunks - 1
        bl = gl % 2
        wait_gather(bl)
        scale(bl)
        fire_wb(gl, bl)
        wait_wb(0)
        wait_wb(1)

    return k


def kernel(tokens, embedding):
    B = tokens.shape[0] * tokens.shape[1]
    idx = tokens.reshape(B // _IDXW, _IDXW).astype(jnp.int32)
    out = _make_sc_kernel(B)(idx, embedding)
    return out.reshape(*tokens.shape, D)
